# Initial kernel scaffold; baseline (speedup 1.0000x reference)
import jax
import jax.numpy as jnp
from jax.experimental import pallas as pl


def kernel(h_subgraph, inter_graph_idx, W_fc0, b_fc0, W_sum0, b_sum0, W_fc1, b_fc1, W_sum1, b_sum1, W_f1, b_f1, W_f2, b_f2):
    raise NotImplementedError("write your pallas kernel here")


# speedup vs baseline: 2.1039x; 2.1039x over previous
import functools

import jax
import jax.numpy as jnp
from jax import lax
from jax.experimental import pallas as pl
from jax.experimental.pallas import tpu as pltpu
from jax.experimental.pallas import tpu_sc as plsc

NSEG = 4096
D = 128
NC, NS, L = 2, 16, 16
NW = NC * NS
BLK = 128
ACC = NSEG + 128
ACC_PW = ACC // NS

_mesh = lambda: plsc.VectorSubcoreMesh(
    core_axis_name="c", subcore_axis_name="s", num_cores=NC, num_subcores=NS)


def _zero_rows(ref, rows, width):
  def body(r, _):
    for c in range(width // L):
      ref[r, pl.ds(c * L, L)] = jnp.zeros((L,), jnp.float32)
    return 0
  lax.fori_loop(0, rows, body, 0)


def _zero_acc_slice(buf, acc, base):
  pltpu.sync_copy(buf, acc.at[pl.ds(base, BLK)])
  pltpu.sync_copy(buf, acc.at[pl.ds(base + BLK, BLK)])
  rem = ACC_PW - 2 * BLK
  if rem:
    pltpu.sync_copy(buf.at[pl.ds(0, rem)], acc.at[pl.ds(base + 2 * BLK, rem)])


def _make_seg0(n):
  nb_full = n // BLK
  tail = n - nb_full * BLK

  @functools.partial(
      pl.kernel,
      out_type=(jax.ShapeDtypeStruct((NC, ACC, D), jnp.float32),
                jax.ShapeDtypeStruct((NC, ACC, D), jnp.float32)),
      mesh=_mesh(),
      scratch_types=(
          pltpu.VMEM_SHARED((ACC, D), jnp.float32),
          pltpu.VMEM_SHARED((ACC, D), jnp.float32),
          pltpu.VMEM((BLK, D), jnp.float32),
          pltpu.VMEM((BLK, D), jnp.float32),
          pltpu.VMEM((BLK,), jnp.int32),
          pltpu.VMEM((tail if tail else 8,), jnp.int32),
      ),
  )
  def seg0(h_hbm, idx_hbm, sums_hbm, cnts_hbm,
           acc, cacc, hv, ones_v, idx_v, idx_t):
    cid = lax.axis_index("c")
    sid = lax.axis_index("s")
    wid = sid * NC + cid
    _zero_rows(hv, BLK, D)
    def ones_row(r, _):
      for c in range(D // L):
        ones_v[r, pl.ds(c * L, L)] = jnp.ones((L,), jnp.float32)
      return 0
    lax.fori_loop(0, BLK, ones_row, 0)
    _zero_acc_slice(hv, acc, sid * ACC_PW)
    _zero_acc_slice(hv, cacc, sid * ACC_PW)
    plsc.subcore_barrier()

    nblk = (nb_full - wid + NW - 1) // NW

    def step(i, _):
      base = (wid + i * NW) * BLK
      pltpu.sync_copy(idx_hbm.at[pl.ds(base, BLK)], idx_v)
      pltpu.sync_copy(h_hbm.at[pl.ds(base, BLK)], hv)
      pltpu.sync_copy(hv, acc.at[idx_v], add=True)
      pltpu.sync_copy(ones_v, cacc.at[idx_v], add=True)
      return 0
    lax.fori_loop(0, nblk, step, 0)

    if tail:
      @pl.when(wid == NW - 1)
      def _():
        base = nb_full * BLK
        pltpu.sync_copy(idx_hbm.at[pl.ds(base, tail)], idx_t)
        pltpu.sync_copy(h_hbm.at[pl.ds(base, tail)], hv.at[pl.ds(0, tail)])
        pltpu.sync_copy(hv.at[pl.ds(0, tail)], acc.at[idx_t], add=True)
        pltpu.sync_copy(ones_v.at[pl.ds(0, tail)], cacc.at[idx_t], add=True)

    plsc.subcore_barrier()
    sl = pl.ds(sid * ACC_PW, ACC_PW)
    pltpu.sync_copy(acc.at[sl], sums_hbm.at[cid, sl])
    pltpu.sync_copy(cacc.at[sl], cnts_hbm.at[cid, sl])

  return seg0


def _make_fuse(npad, write_u):
  blocks_pw = npad // (NW * BLK)

  out_type = [jax.ShapeDtypeStruct((NC, ACC, D), jnp.float32)]
  if write_u:
    out_type.append(jax.ShapeDtypeStruct((npad, D), jnp.float32))

  @functools.partial(
      pl.kernel,
      out_type=tuple(out_type),
      mesh=_mesh(),
      scratch_types=(
          pltpu.VMEM_SHARED((ACC, D), jnp.float32),
          pltpu.VMEM((BLK, D), jnp.float32),
          pltpu.VMEM((BLK, D), jnp.float32),
          pltpu.VMEM((BLK, D), jnp.float32),
          pltpu.VMEM((BLK,), jnp.int32),
          pltpu.SemaphoreType.DMA,
      ),
  )
  def fuse(x1_hbm, x2_hbm, idx_hbm, *rest):
    if write_u:
      sums_hbm, u_hbm, acc, x1v, gv, uv, idx_v, sem = rest
    else:
      sums_hbm, acc, x1v, gv, uv, idx_v, sem = rest
    cid = lax.axis_index("c")
    sid = lax.axis_index("s")
    wid = sid * NC + cid
    _zero_rows(uv, BLK, D)
    _zero_acc_slice(uv, acc, sid * ACC_PW)
    plsc.subcore_barrier()

    def step(i, _):
      base = (wid * blocks_pw + i) * BLK
      pltpu.sync_copy(idx_hbm.at[pl.ds(base, BLK)], idx_v)
      cp = pltpu.async_copy(x2_hbm.at[idx_v], gv, sem)
      pltpu.sync_copy(x1_hbm.at[pl.ds(base, BLK)], x1v)
      cp.wait()

      def row(r, _):
        for c in range(D // L):
          sl = pl.ds(c * L, L)
          t = x1v[r, sl] + gv[r, sl]
          uv[r, sl] = jnp.where(t > 0.0, t, jnp.exp(t) - 1.0)
        return 0
      lax.fori_loop(0, BLK, row, 0)

      if write_u:
        pltpu.sync_copy(uv, u_hbm.at[pl.ds(base, BLK)])
      pltpu.sync_copy(uv, acc.at[idx_v], add=True)
      return 0
    lax.fori_loop(0, blocks_pw, step, 0)

    plsc.subcore_barrier()
    sl = pl.ds(sid * ACC_PW, ACC_PW)
    pltpu.sync_copy(acc.at[sl], sums_hbm.at[cid, sl])

  return fuse


def _mm(h, W, b, npad):
  n = h.shape[0]
  rows = npad // 32

  def body(h_ref, w_ref, b_ref, o_ref):
    o_ref[...] = jnp.dot(h_ref[...], w_ref[...],
                         preferred_element_type=jnp.float32) + b_ref[...]

  return pl.pallas_call(
      body,
      grid=(32,),
      in_specs=[pl.BlockSpec((rows, D), lambda i: (i, 0)),
                pl.BlockSpec((D, D), lambda i: (0, 0)),
                pl.BlockSpec((D,), lambda i: (0,))],
      out_specs=pl.BlockSpec((rows, D), lambda i: (i, 0)),
      out_shape=jax.ShapeDtypeStruct((npad, D), jnp.float32),
  )(h, W, b)


def _x2_body(s_ref, c_ref, w_ref, b_ref, o_ref):
  s = s_ref[0] + s_ref[1]
  c = c_ref[0, :, 0:1] + c_ref[1, :, 0:1]
  mean = s / jnp.maximum(c, 1.0)
  o_ref[...] = jnp.dot(mean, w_ref[...],
                       preferred_element_type=jnp.float32) + b_ref[...]


def _x2(sums, cnts, W, b):
  return pl.pallas_call(
      _x2_body,
      out_shape=jax.ShapeDtypeStruct((ACC, D), jnp.float32),
  )(sums, cnts, W, b)


def _fin_body(s_ref, c_ref, w1_ref, b1_ref, w2_ref, b2_ref, o_ref):
  s = s_ref[0, :NSEG] + s_ref[1, :NSEG]
  c = c_ref[0, :NSEG, 0:1] + c_ref[1, :NSEG, 0:1]
  mean = s / jnp.maximum(c, 1.0)
  z = jnp.maximum(
      jnp.dot(mean, w1_ref[...], preferred_element_type=jnp.float32)
      + b1_ref[...], 0.0)
  o_ref[...] = jnp.dot(z, w2_ref[...],
                       preferred_element_type=jnp.float32) + b2_ref[...]


def _fin(sums, cnts, W1, b1, W2, b2):
  return pl.pallas_call(
      _fin_body,
      out_shape=jax.ShapeDtypeStruct((NSEG, W2.shape[1]), jnp.float32),
  )(sums, cnts, W1, b1, W2, b2)


def kernel(h_subgraph, inter_graph_idx, W_fc0, b_fc0, W_sum0, b_sum0,
           W_fc1, b_fc1, W_sum1, b_sum1, W_f1, b_f1, W_f2, b_f2):
  n = h_subgraph.shape[0]
  chunk = NW * BLK
  npad = ((n + chunk - 1) // chunk) * chunk
  idx_pad = jnp.concatenate(
      [inter_graph_idx,
       jnp.full((npad - n,), NSEG, jnp.int32)])

  sums0, cnts = _make_seg0(n)(h_subgraph, inter_graph_idx)
  x1_0 = _mm(h_subgraph, W_fc0, b_fc0, npad)
  x2_0 = _x2(sums0, cnts, W_sum0, b_sum0)
  sums1, u1 = _make_fuse(npad, True)(x1_0, x2_0, idx_pad)
  x1_1 = _mm(u1, W_fc1, b_fc1, npad)
  x2_1 = _x2(sums1, cnts, W_sum1, b_sum1)
  (sums2,) = _make_fuse(npad, False)(x1_1, x2_1, idx_pad)
  return _fin(sums2, cnts, W_f1, b_f1, W_f2, b_f2)


# speedup vs baseline: 2.2827x; 1.0850x over previous
import functools

import jax
import jax.numpy as jnp
from jax import lax
from jax.experimental import pallas as pl
from jax.experimental.pallas import tpu as pltpu
from jax.experimental.pallas import tpu_sc as plsc

NSEG = 4096
D = 128
NC, NS, L = 2, 16, 16
NW = NC * NS
BLK = 128
ACC = NSEG + 128
ACC_PW = ACC // NS

_mesh = lambda: plsc.VectorSubcoreMesh(
    core_axis_name="c", subcore_axis_name="s", num_cores=NC, num_subcores=NS)


def _zero_rows(ref, rows, width):
  def body(r, _):
    for c in range(width // L):
      ref[r, pl.ds(c * L, L)] = jnp.zeros((L,), jnp.float32)
    return 0
  lax.fori_loop(0, rows, body, 0)


def _zero_acc_slice(buf, acc, base):
  pltpu.sync_copy(buf, acc.at[pl.ds(base, BLK)])
  pltpu.sync_copy(buf, acc.at[pl.ds(base + BLK, BLK)])
  rem = ACC_PW - 2 * BLK
  if rem:
    pltpu.sync_copy(buf.at[pl.ds(0, rem)], acc.at[pl.ds(base + 2 * BLK, rem)])


def _make_seg0(n):
  nb_full = n // BLK
  tail = n - nb_full * BLK
  min_blocks = nb_full // NW
  extra = nb_full % NW
  assert min_blocks >= 2 and min_blocks % 2 == 0

  @functools.partial(
      pl.kernel,
      out_type=(jax.ShapeDtypeStruct((NC, ACC, D), jnp.float32),
                jax.ShapeDtypeStruct((NC, ACC, D), jnp.float32)),
      mesh=_mesh(),
      scratch_types=(
          pltpu.VMEM_SHARED((ACC, D), jnp.float32),
          pltpu.VMEM_SHARED((ACC, D), jnp.float32),
          pltpu.VMEM((BLK, D), jnp.float32),
          pltpu.VMEM((BLK, D), jnp.float32),
          pltpu.VMEM((BLK, D), jnp.float32),
          pltpu.VMEM((BLK,), jnp.int32),
          pltpu.VMEM((BLK,), jnp.int32),
          pltpu.VMEM((tail if tail else 8,), jnp.int32),
          pltpu.SemaphoreType.DMA,
          pltpu.SemaphoreType.DMA,
          pltpu.SemaphoreType.DMA,
          pltpu.SemaphoreType.DMA,
          pltpu.SemaphoreType.DMA,
          pltpu.SemaphoreType.DMA,
      ),
  )
  def seg0(h_hbm, idx_hbm, sums_hbm, cnts_hbm,
           acc, cacc, hv0, hv1, ones_v, idx0, idx1, idx_t,
           sh0, sh1, ss0, ss1, so0, so1):
    cid = lax.axis_index("c")
    sid = lax.axis_index("s")
    wid = sid * NC + cid
    hv = (hv0, hv1)
    idxv = (idx0, idx1)
    sem_h = (sh0, sh1)
    sem_s = (ss0, ss1)
    sem_o = (so0, so1)

    _zero_rows(hv0, BLK, D)
    def ones_row(r, _):
      for c in range(D // L):
        ones_v[r, pl.ds(c * L, L)] = jnp.ones((L,), jnp.float32)
      return 0
    lax.fori_loop(0, BLK, ones_row, 0)
    _zero_acc_slice(hv0, acc, sid * ACC_PW)
    _zero_acc_slice(hv0, cacc, sid * ACC_PW)
    plsc.subcore_barrier()

    def base_of(bl):
      return (wid + bl * NW) * BLK

    def prefetch(bl, s):
      pltpu.sync_copy(idx_hbm.at[pl.ds(base_of(bl), BLK)], idxv[s])
      pltpu.async_copy(h_hbm.at[pl.ds(base_of(bl), BLK)], hv[s], sem_h[s])

    def wait_in(s):
      pltpu.make_async_copy(h_hbm.at[pl.ds(0, BLK)], hv[s], sem_h[s]).wait()

    def emit(s):
      pltpu.async_copy(hv[s], acc.at[idxv[s]], sem_s[s], add=True)
      pltpu.async_copy(ones_v, cacc.at[idxv[s]], sem_o[s], add=True)

    def wait_out(s):
      pltpu.make_async_copy(hv[s], acc.at[idxv[s]], sem_s[s]).wait()
      pltpu.make_async_copy(ones_v, cacc.at[idxv[s]], sem_o[s]).wait()

    prefetch(0, 0)

    def it(i2, _):
      wait_in(0)
      emit(0)
      @pl.when(i2 > 0)
      def _():
        wait_out(1)
      prefetch(2 * i2 + 1, 1)
      wait_in(1)
      emit(1)
      wait_out(0)
      @pl.when((i2 < min_blocks // 2 - 1) | (wid < extra))
      def _():
        prefetch(2 * i2 + 2, 0)
      return 0
    lax.fori_loop(0, min_blocks // 2, it, 0)

    if extra:
      @pl.when(wid < extra)
      def _():
        wait_in(0)
        emit(0)
        wait_out(0)
    wait_out(1)

    if tail:
      @pl.when(wid == NW - 1)
      def _():
        base = nb_full * BLK
        pltpu.sync_copy(idx_hbm.at[pl.ds(base, tail)], idx_t)
        pltpu.sync_copy(h_hbm.at[pl.ds(base, tail)], hv0.at[pl.ds(0, tail)])
        pltpu.sync_copy(hv0.at[pl.ds(0, tail)], acc.at[idx_t], add=True)
        pltpu.sync_copy(ones_v.at[pl.ds(0, tail)], cacc.at[idx_t], add=True)

    plsc.subcore_barrier()
    sl = pl.ds(sid * ACC_PW, ACC_PW)
    pltpu.sync_copy(acc.at[sl], sums_hbm.at[cid, sl])
    pltpu.sync_copy(cacc.at[sl], cnts_hbm.at[cid, sl])

  return seg0


def _make_fuse(npad, write_u):
  blocks_pw = npad // (NW * BLK)
  assert blocks_pw >= 3 and blocks_pw % 2 == 1

  out_type = [jax.ShapeDtypeStruct((NC, ACC, D), jnp.float32)]
  if write_u:
    out_type.append(jax.ShapeDtypeStruct((npad, D), jnp.float32))

  scratch = [
      pltpu.VMEM_SHARED((ACC, D), jnp.float32),
      pltpu.VMEM((BLK, D), jnp.float32),
      pltpu.VMEM((BLK, D), jnp.float32),
      pltpu.VMEM((BLK, D), jnp.float32),
      pltpu.VMEM((BLK, D), jnp.float32),
      pltpu.VMEM((BLK,), jnp.int32),
      pltpu.VMEM((BLK,), jnp.int32),
  ] + [pltpu.SemaphoreType.DMA] * (8 if write_u else 6)

  @functools.partial(
      pl.kernel, out_type=tuple(out_type), mesh=_mesh(),
      scratch_types=tuple(scratch))
  def fuse(x1_hbm, x2_hbm, idx_hbm, *rest):
    if write_u:
      (sums_hbm, u_hbm, acc, a0, a1, g0, g1, idx0, idx1,
       sg0, sg1, sx0, sx1, ss0, ss1, sw0, sw1) = rest
      sem_w = (sw0, sw1)
    else:
      (sums_hbm, acc, a0, a1, g0, g1, idx0, idx1,
       sg0, sg1, sx0, sx1, ss0, ss1) = rest
    cid = lax.axis_index("c")
    sid = lax.axis_index("s")
    wid = sid * NC + cid
    av = (a0, a1)
    gv = (g0, g1)
    idxv = (idx0, idx1)
    sem_g = (sg0, sg1)
    sem_x = (sx0, sx1)
    sem_s = (ss0, ss1)

    _zero_rows(g0, BLK, D)
    _zero_acc_slice(g0, acc, sid * ACC_PW)
    plsc.subcore_barrier()

    def base_of(bl):
      return (wid * blocks_pw + bl) * BLK

    def prefetch(bl, s):
      pltpu.sync_copy(idx_hbm.at[pl.ds(base_of(bl), BLK)], idxv[s])
      pltpu.async_copy(x2_hbm.at[idxv[s]], gv[s], sem_g[s])
      pltpu.async_copy(x1_hbm.at[pl.ds(base_of(bl), BLK)], av[s], sem_x[s])

    def wait_in(s):
      pltpu.make_async_copy(x2_hbm.at[idxv[s]], gv[s], sem_g[s]).wait()
      pltpu.make_async_copy(x1_hbm.at[pl.ds(0, BLK)], av[s], sem_x[s]).wait()

    def compute(s):
      a, g = av[s], gv[s]
      def rows(r8, _):
        for rr in range(8):
          r = r8 * 8 + rr
          for c in range(D // L):
            sl = pl.ds(c * L, L)
            t = a[r, sl] + g[r, sl]
            a[r, sl] = jnp.where(t > 0.0, t, jnp.exp(t) - 1.0)
        return 0
      lax.fori_loop(0, BLK // 8, rows, 0)

    def emit(bl, s):
      if write_u:
        pltpu.async_copy(av[s], u_hbm.at[pl.ds(base_of(bl), BLK)], sem_w[s])
      pltpu.async_copy(av[s], acc.at[idxv[s]], sem_s[s], add=True)

    def wait_out(s):
      if write_u:
        pltpu.make_async_copy(av[s], u_hbm.at[pl.ds(0, BLK)], sem_w[s]).wait()
      pltpu.make_async_copy(av[s], acc.at[idxv[s]], sem_s[s]).wait()

    prefetch(0, 0)

    def it(i2, _):
      b0 = 2 * i2
      wait_in(0)
      compute(0)
      emit(b0, 0)
      @pl.when(i2 > 0)
      def _():
        wait_out(1)
      prefetch(b0 + 1, 1)
      wait_in(1)
      compute(1)
      emit(b0 + 1, 1)
      wait_out(0)
      prefetch(b0 + 2, 0)
      return 0
    lax.fori_loop(0, (blocks_pw - 1) // 2, it, 0)

    wait_in(0)
    compute(0)
    emit(blocks_pw - 1, 0)
    wait_out(0)
    wait_out(1)

    plsc.subcore_barrier()
    sl = pl.ds(sid * ACC_PW, ACC_PW)
    pltpu.sync_copy(acc.at[sl], sums_hbm.at[cid, sl])

  return fuse


def _mm(h, W, b, npad):
  rows = npad // 32

  def body(h_ref, w_ref, b_ref, o_ref):
    o_ref[...] = jnp.dot(h_ref[...], w_ref[...],
                         preferred_element_type=jnp.float32) + b_ref[...]

  return pl.pallas_call(
      body,
      grid=(32,),
      in_specs=[pl.BlockSpec((rows, D), lambda i: (i, 0)),
                pl.BlockSpec((D, D), lambda i: (0, 0)),
                pl.BlockSpec((D,), lambda i: (0,))],
      out_specs=pl.BlockSpec((rows, D), lambda i: (i, 0)),
      out_shape=jax.ShapeDtypeStruct((npad, D), jnp.float32),
  )(h, W, b)


def _x2_body(s_ref, c_ref, w_ref, b_ref, o_ref):
  s = s_ref[0] + s_ref[1]
  c = c_ref[0, :, 0:1] + c_ref[1, :, 0:1]
  mean = s / jnp.maximum(c, 1.0)
  o_ref[...] = jnp.dot(mean, w_ref[...],
                       preferred_element_type=jnp.float32) + b_ref[...]


def _x2(sums, cnts, W, b):
  return pl.pallas_call(
      _x2_body,
      out_shape=jax.ShapeDtypeStruct((ACC, D), jnp.float32),
  )(sums, cnts, W, b)


def _fin_body(s_ref, c_ref, w1_ref, b1_ref, w2_ref, b2_ref, o_ref):
  s = s_ref[0, :NSEG] + s_ref[1, :NSEG]
  c = c_ref[0, :NSEG, 0:1] + c_ref[1, :NSEG, 0:1]
  mean = s / jnp.maximum(c, 1.0)
  z = jnp.maximum(
      jnp.dot(mean, w1_ref[...], preferred_element_type=jnp.float32)
      + b1_ref[...], 0.0)
  o_ref[...] = jnp.dot(z, w2_ref[...],
                       preferred_element_type=jnp.float32) + b2_ref[...]


def _fin(sums, cnts, W1, b1, W2, b2):
  return pl.pallas_call(
      _fin_body,
      out_shape=jax.ShapeDtypeStruct((NSEG, W2.shape[1]), jnp.float32),
  )(sums, cnts, W1, b1, W2, b2)


def kernel(h_subgraph, inter_graph_idx, W_fc0, b_fc0, W_sum0, b_sum0,
           W_fc1, b_fc1, W_sum1, b_sum1, W_f1, b_f1, W_f2, b_f2):
  n = h_subgraph.shape[0]
  chunk = NW * BLK
  npad = ((n + chunk - 1) // chunk) * chunk
  idx_pad = jnp.concatenate(
      [inter_graph_idx,
       jnp.full((npad - n,), NSEG, jnp.int32)])

  sums0, cnts = _make_seg0(n)(h_subgraph, inter_graph_idx)
  x1_0 = _mm(h_subgraph, W_fc0, b_fc0, npad)
  x2_0 = _x2(sums0, cnts, W_sum0, b_sum0)
  sums1, u1 = _make_fuse(npad, True)(x1_0, x2_0, idx_pad)
  x1_1 = _mm(u1, W_fc1, b_fc1, npad)
  x2_1 = _x2(sums1, cnts, W_sum1, b_sum1)
  (sums2,) = _make_fuse(npad, False)(x1_1, x2_1, idx_pad)
  return _fin(sums2, cnts, W_f1, b_f1, W_f2, b_f2)


# speedup vs baseline: 2.7155x; 1.1896x over previous
import functools

import jax
import jax.numpy as jnp
from jax import lax
from jax.experimental import pallas as pl
from jax.experimental.pallas import tpu as pltpu
from jax.experimental.pallas import tpu_sc as plsc

NSEG = 4096
D = 128
NC, NS, L = 2, 16, 16
NW = NC * NS
BLK = 80
ACC = NSEG + 128
ACC_PW = ACC // NS

_mesh = lambda: plsc.VectorSubcoreMesh(
    core_axis_name="c", subcore_axis_name="s", num_cores=NC, num_subcores=NS)


def _zero_rows(ref, rows, width):
  def body(r, _):
    for c in range(width // L):
      ref[r, pl.ds(c * L, L)] = jnp.zeros((L,), jnp.float32)
    return 0
  lax.fori_loop(0, rows, body, 0)


def _zero_acc_slice(buf, acc, base):
  full = ACC_PW // BLK
  for i in range(full):
    pltpu.sync_copy(buf, acc.at[pl.ds(base + i * BLK, BLK)])
  rem = ACC_PW - full * BLK
  if rem:
    pltpu.sync_copy(buf.at[pl.ds(0, rem)], acc.at[pl.ds(base + full * BLK, rem)])


def _make_seg0(n):
  assert n % BLK == 0
  nb = n // BLK
  min_b = nb // NW
  extra = nb % NW
  assert min_b % 3 == 0 and min_b >= 6

  @functools.partial(
      pl.kernel,
      out_type=(jax.ShapeDtypeStruct((NC, ACC, D), jnp.float32),
                jax.ShapeDtypeStruct((NC, ACC, D), jnp.float32)),
      mesh=_mesh(),
      scratch_types=(
          pltpu.VMEM_SHARED((ACC, D), jnp.float32),
          pltpu.VMEM_SHARED((ACC, D), jnp.float32),
          pltpu.VMEM((BLK, D), jnp.float32),
          pltpu.VMEM((BLK, D), jnp.float32),
          pltpu.VMEM((BLK, D), jnp.float32),
          pltpu.VMEM((BLK, D), jnp.float32),
          pltpu.VMEM(((min_b + (1 if extra else 0)) * BLK,), jnp.int32),
          pltpu.VMEM((BLK,), jnp.int32),
          pltpu.VMEM((BLK,), jnp.int32),
          pltpu.VMEM((BLK,), jnp.int32),
          pltpu.SemaphoreType.DMA, pltpu.SemaphoreType.DMA,
          pltpu.SemaphoreType.DMA, pltpu.SemaphoreType.DMA,
          pltpu.SemaphoreType.DMA, pltpu.SemaphoreType.DMA,
          pltpu.SemaphoreType.DMA, pltpu.SemaphoreType.DMA,
          pltpu.SemaphoreType.DMA,
      ),
  )
  def seg0(h_hbm, idx_hbm, sums_hbm, cnts_hbm,
           acc, cacc, hv0, hv1, hv2, ones_v, idxb, ix0, ix1, ix2,
           sh0, sh1, sh2, ss0, ss1, ss2, so0, so1, so2):
    cid = lax.axis_index("c")
    sid = lax.axis_index("s")
    wid = sid * NC + cid
    hv = (hv0, hv1, hv2)
    idx_s = (ix0, ix1, ix2)
    sem_h = (sh0, sh1, sh2)
    sem_s = (ss0, ss1, ss2)
    sem_o = (so0, so1, so2)
    start = wid * min_b + jnp.minimum(wid, extra)

    _zero_rows(hv0, BLK, D)
    def ones_row(r, _):
      for c in range(D // L):
        ones_v[r, pl.ds(c * L, L)] = jnp.ones((L,), jnp.float32)
      return 0
    lax.fori_loop(0, BLK, ones_row, 0)
    _zero_acc_slice(hv0, acc, sid * ACC_PW)
    _zero_acc_slice(hv0, cacc, sid * ACC_PW)
    pltpu.sync_copy(idx_hbm.at[pl.ds(start * BLK, min_b * BLK)],
                    idxb.at[pl.ds(0, min_b * BLK)])
    if extra:
      @pl.when(wid < extra)
      def _():
        pltpu.sync_copy(idx_hbm.at[pl.ds((start + min_b) * BLK, BLK)],
                        idxb.at[pl.ds(min_b * BLK, BLK)])
    plsc.subcore_barrier()

    def prefetch(bl, s):
      for j in range(BLK // L):
        idx_s[s][pl.ds(j * L, L)] = idxb[pl.ds(bl * BLK + j * L, L)]
      pltpu.async_copy(h_hbm.at[pl.ds((start + bl) * BLK, BLK)], hv[s], sem_h[s])

    def wait_in(s):
      pltpu.make_async_copy(h_hbm.at[pl.ds(0, BLK)], hv[s], sem_h[s]).wait()

    def emit(bl, s):
      pltpu.async_copy(hv[s], acc.at[idx_s[s]], sem_s[s], add=True)
      pltpu.async_copy(ones_v, cacc.at[idx_s[s]], sem_o[s], add=True)

    def wait_out(s):
      pltpu.make_async_copy(hv[s], acc.at[idx_s[s]], sem_s[s]).wait()
      pltpu.make_async_copy(ones_v, cacc.at[idx_s[s]], sem_o[s]).wait()

    prefetch(0, 0)
    prefetch(1, 1)
    prefetch(2, 2)

    def it(i3, _):
      for p in range(3):
        b = 3 * i3 + p
        s = p
        wait_in(s)
        emit(b, s)
        if p == 0:
          cond = (i3 > 0)
        elif p == 1:
          cond = (i3 < min_b // 3 - 1) | (wid < extra) if extra else (
              i3 < min_b // 3 - 1)
        else:
          cond = (i3 < min_b // 3 - 1)
        @pl.when(cond)
        def _(b=b, s=s):
          ps = (s + 2) % 3
          wait_out(ps)
          prefetch(b + 2, ps)
      return 0
    lax.fori_loop(0, min_b // 3, it, 0)

    if extra:
      @pl.when(wid < extra)
      def _():
        wait_in(0)
        emit(min_b, 0)
    wait_out(0)
    wait_out(1)
    wait_out(2)

    plsc.subcore_barrier()
    sl = pl.ds(sid * ACC_PW, ACC_PW)
    pltpu.sync_copy(acc.at[sl], sums_hbm.at[cid, sl])
    pltpu.sync_copy(cacc.at[sl], cnts_hbm.at[cid, sl])

  return seg0


def _make_fuse(npad, write_u):
  bpw = npad // (NW * BLK)
  assert bpw % 3 == 1 and bpw >= 7

  out_type = [jax.ShapeDtypeStruct((NC, ACC, D), jnp.float32)]
  if write_u:
    out_type.append(jax.ShapeDtypeStruct((npad, D), jnp.float32))

  scratch = [
      pltpu.VMEM_SHARED((ACC, D), jnp.float32),
      pltpu.VMEM((BLK, D), jnp.float32),
      pltpu.VMEM((BLK, D), jnp.float32),
      pltpu.VMEM((BLK, D), jnp.float32),
      pltpu.VMEM((BLK, D), jnp.float32),
      pltpu.VMEM((BLK, D), jnp.float32),
      pltpu.VMEM((BLK, D), jnp.float32),
      pltpu.VMEM((bpw * BLK,), jnp.int32),
      pltpu.VMEM((BLK,), jnp.int32),
      pltpu.VMEM((BLK,), jnp.int32),
      pltpu.VMEM((BLK,), jnp.int32),
  ] + [pltpu.SemaphoreType.DMA] * (12 if write_u else 9)

  @functools.partial(
      pl.kernel, out_type=tuple(out_type), mesh=_mesh(),
      scratch_types=tuple(scratch))
  def fuse(x1_hbm, x2_hbm, idx_hbm, *rest):
    if write_u:
      (sums_hbm, u_hbm, acc, a0, a1, a2, g0, g1, g2, idxb, ix0, ix1, ix2,
       sg0, sg1, sg2, sx0, sx1, sx2, ss0, ss1, ss2, sw0, sw1, sw2) = rest
      sem_w = (sw0, sw1, sw2)
    else:
      (sums_hbm, acc, a0, a1, a2, g0, g1, g2, idxb, ix0, ix1, ix2,
       sg0, sg1, sg2, sx0, sx1, sx2, ss0, ss1, ss2) = rest
    cid = lax.axis_index("c")
    sid = lax.axis_index("s")
    wid = sid * NC + cid
    av = (a0, a1, a2)
    gv = (g0, g1, g2)
    idx_s = (ix0, ix1, ix2)
    sem_g = (sg0, sg1, sg2)
    sem_x = (sx0, sx1, sx2)
    sem_s = (ss0, ss1, ss2)

    _zero_rows(g0, BLK, D)
    _zero_acc_slice(g0, acc, sid * ACC_PW)
    pltpu.sync_copy(idx_hbm.at[pl.ds(wid * bpw * BLK, bpw * BLK)], idxb)
    plsc.subcore_barrier()

    def prefetch(bl, s):
      for j in range(BLK // L):
        idx_s[s][pl.ds(j * L, L)] = idxb[pl.ds(bl * BLK + j * L, L)]
      pltpu.async_copy(x2_hbm.at[idx_s[s]], gv[s], sem_g[s])
      pltpu.async_copy(x1_hbm.at[pl.ds((wid * bpw + bl) * BLK, BLK)],
                       av[s], sem_x[s])

    def wait_in(s):
      pltpu.make_async_copy(x2_hbm.at[idx_s[s]], gv[s], sem_g[s]).wait()
      pltpu.make_async_copy(x1_hbm.at[pl.ds(0, BLK)], av[s], sem_x[s]).wait()

    def compute(s):
      a, g = av[s], gv[s]
      def rows(r8, _):
        for rr in range(8):
          r = r8 * 8 + rr
          for c in range(D // L):
            sl = pl.ds(c * L, L)
            t = a[r, sl] + g[r, sl]
            a[r, sl] = jnp.where(t > 0.0, t, jnp.exp(t) - 1.0)
        return 0
      lax.fori_loop(0, BLK // 8, rows, 0)

    def emit(bl, s):
      if write_u:
        pltpu.async_copy(av[s], u_hbm.at[pl.ds((wid * bpw + bl) * BLK, BLK)],
                         sem_w[s])
      pltpu.async_copy(av[s], acc.at[idx_s[s]], sem_s[s], add=True)

    def wait_out(s):
      if write_u:
        pltpu.make_async_copy(av[s], u_hbm.at[pl.ds(0, BLK)], sem_w[s]).wait()
      pltpu.make_async_copy(av[s], acc.at[idx_s[s]], sem_s[s]).wait()

    prefetch(0, 0)
    prefetch(1, 1)
    prefetch(2, 2)

    def it(i3, _):
      for p in range(3):
        b = 3 * i3 + p
        s = p
        wait_in(s)
        compute(s)
        emit(b, s)
        if p == 0:
          cond = (i3 > 0)
        elif p == 1:
          cond = True
        else:
          cond = (i3 < (bpw - 1) // 3 - 1)
        if cond is not True:
          @pl.when(cond)
          def _(b=b, s=s):
            ps = (s + 2) % 3
            wait_out(ps)
            prefetch(b + 2, ps)
        else:
          ps = (s + 2) % 3
          wait_out(ps)
          prefetch(b + 2, ps)
      return 0
    lax.fori_loop(0, (bpw - 1) // 3, it, 0)

    wait_in(0)
    compute(0)
    emit(bpw - 1, 0)
    wait_out(1)
    wait_out(2)
    wait_out(0)

    plsc.subcore_barrier()
    sl = pl.ds(sid * ACC_PW, ACC_PW)
    pltpu.sync_copy(acc.at[sl], sums_hbm.at[cid, sl])

  return fuse


def _mm(h, W, b, npad):
  rows = npad // 32

  def body(h_ref, w_ref, b_ref, o_ref):
    o_ref[...] = jnp.dot(h_ref[...], w_ref[...],
                         preferred_element_type=jnp.float32) + b_ref[...]

  return pl.pallas_call(
      body,
      grid=(32,),
      in_specs=[pl.BlockSpec((rows, D), lambda i: (i, 0)),
                pl.BlockSpec((D, D), lambda i: (0, 0)),
                pl.BlockSpec((D,), lambda i: (0,))],
      out_specs=pl.BlockSpec((rows, D), lambda i: (i, 0)),
      out_shape=jax.ShapeDtypeStruct((npad, D), jnp.float32),
  )(h, W, b)


def _x2_body(s_ref, c_ref, w_ref, b_ref, o_ref):
  s = s_ref[0] + s_ref[1]
  c = c_ref[0, :, 0:1] + c_ref[1, :, 0:1]
  mean = s / jnp.maximum(c, 1.0)
  o_ref[...] = jnp.dot(mean, w_ref[...],
                       preferred_element_type=jnp.float32) + b_ref[...]


def _x2(sums, cnts, W, b):
  return pl.pallas_call(
      _x2_body,
      out_shape=jax.ShapeDtypeStruct((ACC, D), jnp.float32),
  )(sums, cnts, W, b)


def _fin_body(s_ref, c_ref, w1_ref, b1_ref, w2_ref, b2_ref, o_ref):
  s = s_ref[0, :NSEG] + s_ref[1, :NSEG]
  c = c_ref[0, :NSEG, 0:1] + c_ref[1, :NSEG, 0:1]
  mean = s / jnp.maximum(c, 1.0)
  z = jnp.maximum(
      jnp.dot(mean, w1_ref[...], preferred_element_type=jnp.float32)
      + b1_ref[...], 0.0)
  o_ref[...] = jnp.dot(z, w2_ref[...],
                       preferred_element_type=jnp.float32) + b2_ref[...]


def _fin(sums, cnts, W1, b1, W2, b2):
  return pl.pallas_call(
      _fin_body,
      out_shape=jax.ShapeDtypeStruct((NSEG, W2.shape[1]), jnp.float32),
  )(sums, cnts, W1, b1, W2, b2)


def kernel(h_subgraph, inter_graph_idx, W_fc0, b_fc0, W_sum0, b_sum0,
           W_fc1, b_fc1, W_sum1, b_sum1, W_f1, b_f1, W_f2, b_f2):
  n = h_subgraph.shape[0]
  chunk = NW * BLK
  npad = ((n + chunk - 1) // chunk) * chunk
  idx_pad = jnp.concatenate(
      [inter_graph_idx, jnp.full((npad - n,), NSEG, jnp.int32)])

  sums0, cnts = _make_seg0(n)(h_subgraph, inter_graph_idx)
  x1_0 = _mm(h_subgraph, W_fc0, b_fc0, npad)
  x2_0 = _x2(sums0, cnts, W_sum0, b_sum0)
  sums1, u1 = _make_fuse(npad, True)(x1_0, x2_0, idx_pad)
  x1_1 = _mm(u1, W_fc1, b_fc1, npad)
  x2_1 = _x2(sums1, cnts, W_sum1, b_sum1)
  (sums2,) = _make_fuse(npad, False)(x1_1, x2_1, idx_pad)
  return _fin(sums2, cnts, W_f1, b_f1, W_f2, b_f2)


# speedup vs baseline: 5.0647x; 1.8651x over previous
import functools

import jax
import jax.numpy as jnp
from jax import lax
from jax.experimental import pallas as pl
from jax.experimental.pallas import tpu as pltpu
from jax.experimental.pallas import tpu_sc as plsc

NSEG = 4096
D = 128
NC, NS, L = 2, 16, 16
NW = NC * NS
BLK = 80
ACC = NSEG + 128
ACC_PW = ACC // NS

_mesh = lambda: plsc.VectorSubcoreMesh(
    core_axis_name="c", subcore_axis_name="s", num_cores=NC, num_subcores=NS)


def _zero_rows(ref, rows, width):
  def body(r, _):
    for c in range(width // L):
      ref[r, pl.ds(c * L, L)] = jnp.zeros((L,), jnp.float32)
    return 0
  lax.fori_loop(0, rows, body, 0)


def _zero_acc_slice(buf, acc, base):
  full = ACC_PW // BLK
  for i in range(full):
    pltpu.sync_copy(buf, acc.at[pl.ds(base + i * BLK, BLK)])
  rem = ACC_PW - full * BLK
  if rem:
    pltpu.sync_copy(buf.at[pl.ds(0, rem)], acc.at[pl.ds(base + full * BLK, rem)])


def _make_seg0(n):
  assert n % BLK == 0
  nb = n // BLK
  min_b = nb // NW
  extra = nb % NW
  assert min_b % 3 == 0 and min_b >= 6

  @functools.partial(
      pl.kernel,
      out_type=(jax.ShapeDtypeStruct((NC, ACC, D), jnp.float32),
                jax.ShapeDtypeStruct((NC, ACC, D), jnp.float32)),
      mesh=_mesh(),
      scratch_types=(
          pltpu.VMEM_SHARED((ACC, D), jnp.float32),
          pltpu.VMEM_SHARED((ACC, D), jnp.float32),
          pltpu.VMEM((BLK, D), jnp.float32),
          pltpu.VMEM((BLK, D), jnp.float32),
          pltpu.VMEM((BLK, D), jnp.float32),
          pltpu.VMEM((BLK, D), jnp.float32),
          pltpu.VMEM(((min_b + (1 if extra else 0)) * BLK,), jnp.int32),
          pltpu.VMEM((BLK,), jnp.int32),
          pltpu.VMEM((BLK,), jnp.int32),
          pltpu.VMEM((BLK,), jnp.int32),
          pltpu.SemaphoreType.DMA, pltpu.SemaphoreType.DMA,
          pltpu.SemaphoreType.DMA, pltpu.SemaphoreType.DMA,
          pltpu.SemaphoreType.DMA, pltpu.SemaphoreType.DMA,
          pltpu.SemaphoreType.DMA, pltpu.SemaphoreType.DMA,
          pltpu.SemaphoreType.DMA,
      ),
  )
  def seg0(h_hbm, idx_hbm, sums_hbm, cnts_hbm,
           acc, cacc, hv0, hv1, hv2, ones_v, idxb, ix0, ix1, ix2,
           sh0, sh1, sh2, ss0, ss1, ss2, so0, so1, so2):
    cid = lax.axis_index("c")
    sid = lax.axis_index("s")
    wid = sid * NC + cid
    hv = (hv0, hv1, hv2)
    idx_s = (ix0, ix1, ix2)
    sem_h = (sh0, sh1, sh2)
    sem_s = (ss0, ss1, ss2)
    sem_o = (so0, so1, so2)
    start = wid * min_b + jnp.minimum(wid, extra)

    _zero_rows(hv0, BLK, D)
    def ones_row(r, _):
      for c in range(D // L):
        ones_v[r, pl.ds(c * L, L)] = jnp.ones((L,), jnp.float32)
      return 0
    lax.fori_loop(0, BLK, ones_row, 0)
    _zero_acc_slice(hv0, acc, sid * ACC_PW)
    _zero_acc_slice(hv0, cacc, sid * ACC_PW)
    pltpu.sync_copy(idx_hbm.at[pl.ds(start * BLK, min_b * BLK)],
                    idxb.at[pl.ds(0, min_b * BLK)])
    if extra:
      @pl.when(wid < extra)
      def _():
        pltpu.sync_copy(idx_hbm.at[pl.ds((start + min_b) * BLK, BLK)],
                        idxb.at[pl.ds(min_b * BLK, BLK)])
    plsc.subcore_barrier()

    def prefetch(bl, s):
      for j in range(BLK // L):
        idx_s[s][pl.ds(j * L, L)] = idxb[pl.ds(bl * BLK + j * L, L)]
      pltpu.async_copy(h_hbm.at[pl.ds((start + bl) * BLK, BLK)], hv[s], sem_h[s])

    def wait_in(s):
      pltpu.make_async_copy(h_hbm.at[pl.ds(0, BLK)], hv[s], sem_h[s]).wait()

    def emit(bl, s):
      pltpu.async_copy(hv[s], acc.at[idx_s[s]], sem_s[s], add=True)
      pltpu.async_copy(ones_v, cacc.at[idx_s[s]], sem_o[s], add=True)

    def wait_out(s):
      pltpu.make_async_copy(hv[s], acc.at[idx_s[s]], sem_s[s]).wait()
      pltpu.make_async_copy(ones_v, cacc.at[idx_s[s]], sem_o[s]).wait()

    prefetch(0, 0)
    prefetch(1, 1)
    prefetch(2, 2)

    def it(i3, _):
      for p in range(3):
        b = 3 * i3 + p
        s = p
        wait_in(s)
        emit(b, s)
        if p == 0:
          cond = (i3 > 0)
        elif p == 1:
          cond = (i3 < min_b // 3 - 1) | (wid < extra) if extra else (
              i3 < min_b // 3 - 1)
        else:
          cond = (i3 < min_b // 3 - 1)
        @pl.when(cond)
        def _(b=b, s=s):
          ps = (s + 2) % 3
          wait_out(ps)
          prefetch(b + 2, ps)
      return 0
    lax.fori_loop(0, min_b // 3, it, 0)

    if extra:
      @pl.when(wid < extra)
      def _():
        wait_in(0)
        emit(min_b, 0)
    wait_out(0)
    wait_out(1)
    wait_out(2)

    plsc.subcore_barrier()
    sl = pl.ds(sid * ACC_PW, ACC_PW)
    pltpu.sync_copy(acc.at[sl], sums_hbm.at[cid, sl])
    pltpu.sync_copy(cacc.at[sl], cnts_hbm.at[cid, sl])

  return seg0


def _make_fuse(npad, write_u):
  bpw = npad // (NW * BLK)
  assert bpw % 6 == 4 and bpw >= 10

  out_type = [jax.ShapeDtypeStruct((NC, ACC, D), jnp.float32)]
  if write_u:
    out_type.append(jax.ShapeDtypeStruct((npad, D), jnp.float32))

  scratch = [
      pltpu.VMEM_SHARED((ACC, D), jnp.float32),
      pltpu.VMEM_SHARED((ACC, D), jnp.float32),
      pltpu.VMEM((BLK, D), jnp.float32),
      pltpu.VMEM((BLK, D), jnp.float32),
      pltpu.VMEM((BLK, D), jnp.float32),
      pltpu.VMEM((BLK, D), jnp.float32),
      pltpu.VMEM((BLK, D), jnp.float32),
      pltpu.VMEM((bpw * BLK,), jnp.int32),
      pltpu.VMEM((BLK,), jnp.int32),
      pltpu.VMEM((BLK,), jnp.int32),
      pltpu.VMEM((BLK,), jnp.int32),
  ] + [pltpu.SemaphoreType.DMA] * (11 if write_u else 8)

  @functools.partial(
      pl.kernel, out_type=tuple(out_type), mesh=_mesh(),
      scratch_types=tuple(scratch))
  def fuse(x1_hbm, x2_hbm, idx_hbm, *rest):
    if write_u:
      (sums_hbm, u_hbm, acc, x2s, a0, a1, a2, g0, g1, idxb, ix0, ix1, ix2,
       sg0, sg1, sx0, sx1, sx2, ss0, ss1, ss2, sw0, sw1, sw2) = rest
      sem_w = (sw0, sw1, sw2)
    else:
      (sums_hbm, acc, x2s, a0, a1, a2, g0, g1, idxb, ix0, ix1, ix2,
       sg0, sg1, sx0, sx1, sx2, ss0, ss1, ss2) = rest
    cid = lax.axis_index("c")
    sid = lax.axis_index("s")
    wid = sid * NC + cid
    av = (a0, a1, a2)
    gv = (g0, g1)
    idx_s = (ix0, ix1, ix2)
    sem_g = (sg0, sg1)
    sem_x = (sx0, sx1, sx2)
    sem_s = (ss0, ss1, ss2)

    _zero_rows(a0, BLK, D)
    _zero_acc_slice(a0, acc, sid * ACC_PW)
    stg = pl.ds(sid * ACC_PW, ACC_PW)
    pltpu.sync_copy(x2_hbm.at[stg], x2s.at[stg])
    pltpu.sync_copy(idx_hbm.at[pl.ds(wid * bpw * BLK, bpw * BLK)], idxb)
    plsc.subcore_barrier()

    def prefetch_x(bl, sa):
      for j in range(BLK // L):
        idx_s[sa][pl.ds(j * L, L)] = idxb[pl.ds(bl * BLK + j * L, L)]
      pltpu.async_copy(x1_hbm.at[pl.ds((wid * bpw + bl) * BLK, BLK)],
                       av[sa], sem_x[sa])

    def gather_issue(sa, sg):
      pltpu.async_copy(x2s.at[idx_s[sa]], gv[sg], sem_g[sg])

    def wait_in(sa, sg):
      pltpu.make_async_copy(x2s.at[idx_s[sa]], gv[sg], sem_g[sg]).wait()
      pltpu.make_async_copy(x1_hbm.at[pl.ds(0, BLK)], av[sa], sem_x[sa]).wait()

    def compute(sa, sg):
      a, g = av[sa], gv[sg]
      def rows(r8, _):
        for rr in range(8):
          r = r8 * 8 + rr
          for c in range(D // L):
            sl = pl.ds(c * L, L)
            t = a[r, sl] + g[r, sl]
            a[r, sl] = jnp.where(t > 0.0, t, jnp.exp(t) - 1.0)
        return 0
      lax.fori_loop(0, BLK // 8, rows, 0)

    def emit(bl, sa):
      if write_u:
        pltpu.async_copy(av[sa], u_hbm.at[pl.ds((wid * bpw + bl) * BLK, BLK)],
                         sem_w[sa])
      pltpu.async_copy(av[sa], acc.at[idx_s[sa]], sem_s[sa], add=True)

    def wait_out(sa):
      if write_u:
        pltpu.make_async_copy(av[sa], u_hbm.at[pl.ds(0, BLK)], sem_w[sa]).wait()
      pltpu.make_async_copy(av[sa], acc.at[idx_s[sa]], sem_s[sa]).wait()

    def stage(b, sa, sg, drain_cond, pf):
      wait_in(sa, sg)
      compute(sa, sg)
      ps = (sa + 2) % 3
      if pf:
        if drain_cond is True:
          wait_out(ps)
        else:
          @pl.when(drain_cond)
          def _():
            wait_out(ps)
        prefetch_x(b + 2, ps)
        gather_issue(ps, sg)
      elif drain_cond is True:
        wait_out(ps)
      emit(b, sa)

    prefetch_x(0, 0)
    gather_issue(0, 0)
    prefetch_x(1, 1)
    gather_issue(1, 1)

    def it(i6, _):
      for p in range(6):
        b = 6 * i6 + p
        stage(b, p % 3, p % 2,
              True if p else (i6 > 0), True)
      return 0
    lax.fori_loop(0, (bpw - 4) // 6, it, 0)

    base = bpw - 4
    stage(base + 0, (base + 0) % 3, (base + 0) % 2, True, True)
    stage(base + 1, (base + 1) % 3, (base + 1) % 2, True, True)
    stage(base + 2, (base + 2) % 3, (base + 2) % 2, True, False)
    stage(base + 3, (base + 3) % 3, (base + 3) % 2, True, False)
    wait_out((base + 3) % 3)

    plsc.subcore_barrier()
    sl = pl.ds(sid * ACC_PW, ACC_PW)
    pltpu.sync_copy(acc.at[sl], sums_hbm.at[cid, sl])

  return fuse


def _mm(h, W, b, npad):
  rows = npad // 32

  def body(h_ref, w_ref, b_ref, o_ref):
    o_ref[...] = jnp.dot(h_ref[...], w_ref[...],
                         preferred_element_type=jnp.float32) + b_ref[...]

  return pl.pallas_call(
      body,
      grid=(32,),
      in_specs=[pl.BlockSpec((rows, D), lambda i: (i, 0)),
                pl.BlockSpec((D, D), lambda i: (0, 0)),
                pl.BlockSpec((D,), lambda i: (0,))],
      out_specs=pl.BlockSpec((rows, D), lambda i: (i, 0)),
      out_shape=jax.ShapeDtypeStruct((npad, D), jnp.float32),
  )(h, W, b)


def _x2_body(s_ref, c_ref, w_ref, b_ref, o_ref):
  s = s_ref[0] + s_ref[1]
  c = c_ref[0, :, 0:1] + c_ref[1, :, 0:1]
  mean = s / jnp.maximum(c, 1.0)
  o_ref[...] = jnp.dot(mean, w_ref[...],
                       preferred_element_type=jnp.float32) + b_ref[...]


def _x2(sums, cnts, W, b):
  return pl.pallas_call(
      _x2_body,
      out_shape=jax.ShapeDtypeStruct((ACC, D), jnp.float32),
  )(sums, cnts, W, b)


def _fin_body(s_ref, c_ref, w1_ref, b1_ref, w2_ref, b2_ref, o_ref):
  s = s_ref[0, :NSEG] + s_ref[1, :NSEG]
  c = c_ref[0, :NSEG, 0:1] + c_ref[1, :NSEG, 0:1]
  mean = s / jnp.maximum(c, 1.0)
  z = jnp.maximum(
      jnp.dot(mean, w1_ref[...], preferred_element_type=jnp.float32)
      + b1_ref[...], 0.0)
  o_ref[...] = jnp.dot(z, w2_ref[...],
                       preferred_element_type=jnp.float32) + b2_ref[...]


def _fin(sums, cnts, W1, b1, W2, b2):
  return pl.pallas_call(
      _fin_body,
      out_shape=jax.ShapeDtypeStruct((NSEG, W2.shape[1]), jnp.float32),
  )(sums, cnts, W1, b1, W2, b2)


def kernel(h_subgraph, inter_graph_idx, W_fc0, b_fc0, W_sum0, b_sum0,
           W_fc1, b_fc1, W_sum1, b_sum1, W_f1, b_f1, W_f2, b_f2):
  n = h_subgraph.shape[0]
  chunk = NW * BLK
  npad = ((n + chunk - 1) // chunk) * chunk
  idx_pad = jnp.concatenate(
      [inter_graph_idx, jnp.full((npad - n,), NSEG, jnp.int32)])

  sums0, cnts = _make_seg0(n)(h_subgraph, inter_graph_idx)
  x1_0 = _mm(h_subgraph, W_fc0, b_fc0, npad)
  x2_0 = _x2(sums0, cnts, W_sum0, b_sum0)
  sums1, u1 = _make_fuse(npad, True)(x1_0, x2_0, idx_pad)
  x1_1 = _mm(u1, W_fc1, b_fc1, npad)
  x2_1 = _x2(sums1, cnts, W_sum1, b_sum1)
  (sums2,) = _make_fuse(npad, False)(x1_1, x2_1, idx_pad)
  return _fin(sums2, cnts, W_f1, b_f1, W_f2, b_f2)
